# Initial kernel scaffold; baseline (speedup 1.0000x reference)
import jax
import jax.numpy as jnp
from jax.experimental import pallas as pl


def kernel(x, edge_index, edge_weight, conv_W, conv_b, peep, gate_b, lw1, lb1, lw2, lb2, lw3, lb3, lw4, lb4):
    raise NotImplementedError("write your pallas kernel here")


# speedup vs baseline: 4.9474x; 4.9474x over previous
import functools

import jax
import jax.numpy as jnp
from jax import lax
from jax.experimental import pallas as pl
from jax.experimental.pallas import tpu as pltpu
from jax.experimental.pallas import tpu_sc as plsc

N = 10000
D = 128
K = 5
E = 320000
NC, NS = 2, 16
NW = NC * NS
EPW = 10240
EP = EPW * NW
CH = 128
NCH = EPW // CH
HC = 40
ZC = 16
NZC = N // ZC

_mesh = plsc.VectorSubcoreMesh(
    core_axis_name="c", subcore_axis_name="s", num_cores=NC, num_subcores=NS)


@functools.partial(
    pl.kernel,
    out_type=jax.ShapeDtypeStruct((NW, 1, N), jnp.float32),
    mesh=_mesh,
    compiler_params=pltpu.CompilerParams(needs_layout_passes=False),
    scratch_types=[
        pltpu.VMEM((NCH, CH), jnp.int32),
        pltpu.VMEM((NCH, CH), jnp.float32),
        pltpu.VMEM((N,), jnp.float32),
    ],
)
def _deg_kernel(src_hbm, w_hbm, out_hbm, src_v, w_v, deg_v):
    c = lax.axis_index("c")
    s = lax.axis_index("s")
    wid = c * NS + s
    pltpu.sync_copy(src_hbm.at[wid], src_v)
    pltpu.sync_copy(w_hbm.at[wid], w_v)

    def zbody(i, _):
        deg_v[pl.ds(i * 16, 16)] = jnp.zeros((16,), jnp.float32)
        return 0
    lax.fori_loop(0, N // 16, zbody, 0)

    def ebody(k, _):
        for g in range(CH // 16):
            sl = pl.ds(g * 16, 16)
            plsc.addupdate_scatter(deg_v, [src_v[k, sl]], w_v[k, sl])
        return 0
    lax.fori_loop(0, NCH, ebody, 0)
    pltpu.sync_copy(deg_v, out_hbm.at[wid, 0])


@functools.partial(
    pl.kernel,
    out_type=jax.ShapeDtypeStruct((NW, NCH, CH), jnp.float32),
    mesh=_mesh,
    compiler_params=pltpu.CompilerParams(needs_layout_passes=False),
    scratch_types=[
        pltpu.VMEM((N,), jnp.float32),
        pltpu.VMEM((NCH, CH), jnp.int32),
        pltpu.VMEM((NCH, CH), jnp.int32),
        pltpu.VMEM((NCH, CH), jnp.float32),
        pltpu.VMEM((NCH, CH), jnp.float32),
    ],
)
def _nw_kernel(dis_hbm, src_hbm, dst_hbm, w_hbm, out_hbm,
               dis_v, src_v, dst_v, w_v, nw_v):
    c = lax.axis_index("c")
    s = lax.axis_index("s")
    wid = c * NS + s
    pltpu.sync_copy(dis_hbm, dis_v)
    pltpu.sync_copy(src_hbm.at[wid], src_v)
    pltpu.sync_copy(dst_hbm.at[wid], dst_v)
    pltpu.sync_copy(w_hbm.at[wid], w_v)

    def ebody(k, _):
        for g in range(CH // 16):
            sl = pl.ds(g * 16, 16)
            a = plsc.load_gather(dis_v, [src_v[k, sl]])
            b = plsc.load_gather(dis_v, [dst_v[k, sl]])
            nw_v[k, sl] = -(a * w_v[k, sl] * b)
        return 0
    lax.fori_loop(0, NCH, ebody, 0)
    pltpu.sync_copy(nw_v, out_hbm.at[wid])


@functools.partial(
    pl.kernel,
    out_type=jax.ShapeDtypeStruct((NC, N, D), jnp.float32),
    mesh=_mesh,
    compiler_params=pltpu.CompilerParams(needs_layout_passes=False),
    scratch_types=[
        pltpu.VMEM((HC, CH), jnp.int32),
        pltpu.VMEM((HC, CH), jnp.int32),
        pltpu.VMEM((HC, CH), jnp.float32),
        pltpu.VMEM((CH, D), jnp.float32),
        pltpu.VMEM((CH, D), jnp.float32),
        pltpu.VMEM((ZC, D), jnp.float32),
        pltpu.VMEM_SHARED((N, D), jnp.float32),
        pltpu.SemaphoreType.DMA,
        pltpu.SemaphoreType.DMA,
    ],
)
def _prop_kernel(h_hbm, src_hbm, dst_hbm, nw_hbm, out_hbm,
                 src_v, dst_v, nw_v, buf0, buf1, zbuf, acc, sem0, sem1):
    c = lax.axis_index("c")
    s = lax.axis_index("s")
    wid = c * NS + s

    def load_idx(p):
        pltpu.sync_copy(src_hbm.at[wid, pl.ds(p * HC, HC)], src_v)
        pltpu.sync_copy(dst_hbm.at[wid, pl.ds(p * HC, HC)], dst_v)
        pltpu.sync_copy(nw_hbm.at[wid, pl.ds(p * HC, HC)], nw_v)

    load_idx(0)
    pltpu.async_copy(h_hbm.at[src_v.at[0]], buf0, sem0)

    def zb(i, _):
        for f in range(D // 16):
            zbuf[i, pl.ds(f * 16, 16)] = jnp.zeros((16,), jnp.float32)
        return 0
    lax.fori_loop(0, ZC, zb, 0)
    for t in range(NZC // NS + 1):
        cid = s + NS * t

        @pl.when(cid < NZC)
        def _():
            pltpu.sync_copy(zbuf, acc.at[pl.ds(cid * ZC, ZC)])
    plsc.subcore_barrier()

    def scale_scatter(buf, k):
        def eb(g, _):
            w16 = nw_v[k, pl.ds(g * 16, 16)]
            for i in range(16):
                e = g * 16 + i
                sv = jnp.full((16,), w16[i], jnp.float32)
                for f in range(D // 16):
                    sl = pl.ds(f * 16, 16)
                    buf[e, sl] = buf[e, sl] * sv
            return 0
        lax.fori_loop(0, CH // 16, eb, 0)
        pltpu.sync_copy(buf, acc.at[dst_v.at[k]], add=True)

    def body(j, _):
        k0 = 2 * j
        k1 = 2 * j + 1
        pltpu.make_async_copy(h_hbm.at[src_v.at[k0]], buf0, sem0).wait()
        pltpu.async_copy(h_hbm.at[src_v.at[k1]], buf1, sem1)
        scale_scatter(buf0, k0)
        pltpu.make_async_copy(h_hbm.at[src_v.at[k1]], buf1, sem1).wait()

        @pl.when(k1 + 1 < HC)
        def _():
            pltpu.async_copy(h_hbm.at[src_v.at[k1 + 1]], buf0, sem0)
        scale_scatter(buf1, k1)
        return 0

    lax.fori_loop(0, HC // 2, body, 0)
    load_idx(1)
    pltpu.async_copy(h_hbm.at[src_v.at[0]], buf0, sem0)
    lax.fori_loop(0, HC // 2, body, 0)

    plsc.subcore_barrier()
    for t in range(NZC // NS + 1):
        cid = s + NS * t

        @pl.when(cid < NZC)
        def _():
            pltpu.sync_copy(acc.at[pl.ds(cid * ZC, ZC)],
                            out_hbm.at[c, pl.ds(cid * ZC, ZC)])


R = 400


def _dis_body(degp_ref, o_ref):
    deg = jnp.sum(degp_ref[...], axis=0, keepdims=True)
    o_ref[...] = jnp.where(deg > 0, 1.0 / jnp.sqrt(deg), 0.0)


def _comb1_body(p_ref, o_ref):
    o_ref[...] = p_ref[0] + p_ref[1]


def _comb2_body(p_ref, tp_ref, o_ref):
    o_ref[...] = 2.0 * (p_ref[0] + p_ref[1]) - tp_ref[...]


def _gates_body(t0, t1, t2, t3, t4, w_ref, b_ref, o_ref):
    acc = jnp.dot(t0[...], w_ref[0], preferred_element_type=jnp.float32)
    acc = acc + jnp.dot(t1[...], w_ref[1], preferred_element_type=jnp.float32)
    acc = acc + jnp.dot(t2[...], w_ref[2], preferred_element_type=jnp.float32)
    acc = acc + jnp.dot(t3[...], w_ref[3], preferred_element_type=jnp.float32)
    acc = acc + jnp.dot(t4[...], w_ref[4], preferred_element_type=jnp.float32)
    gi = jax.nn.sigmoid(acc[:, 0:D] + b_ref[0:1, :])
    gt = jnp.tanh(acc[:, D:2 * D] + b_ref[1:2, :])
    cc = gi * gt
    go = jax.nn.sigmoid(acc[:, 2 * D:3 * D] + b_ref[2:3, :] + b_ref[3:4, :] * cc)
    o_ref[...] = jnp.maximum(go * jnp.tanh(cc), 0.0)


def _head_body(h_ref, w1, b1, w2, b2, w3, b3, w4, b4, o_ref):
    v = jnp.dot(h_ref[...], w1[...], preferred_element_type=jnp.float32) + b1[...]
    v = jnp.dot(v, w2[...], preferred_element_type=jnp.float32) + b2[...]
    v = jnp.dot(v, w3[...], preferred_element_type=jnp.float32) + b3[...]
    o_ref[...] = jnp.dot(v, w4[...], preferred_element_type=jnp.float32) + b4[...]


_dis_call = pl.pallas_call(
    _dis_body, out_shape=jax.ShapeDtypeStruct((1, N), jnp.float32))

_comb1 = pl.pallas_call(
    _comb1_body,
    grid=(N // R,),
    in_specs=[pl.BlockSpec((2, R, D), lambda i: (0, i, 0))],
    out_specs=pl.BlockSpec((R, D), lambda i: (i, 0)),
    out_shape=jax.ShapeDtypeStruct((N, D), jnp.float32))

_comb2 = pl.pallas_call(
    _comb2_body,
    grid=(N // R,),
    in_specs=[pl.BlockSpec((2, R, D), lambda i: (0, i, 0)),
              pl.BlockSpec((R, D), lambda i: (i, 0))],
    out_specs=pl.BlockSpec((R, D), lambda i: (i, 0)),
    out_shape=jax.ShapeDtypeStruct((N, D), jnp.float32))

_gates = pl.pallas_call(
    _gates_body,
    grid=(N // R,),
    in_specs=[pl.BlockSpec((R, D), lambda i: (i, 0))] * 5
    + [pl.BlockSpec((K, D, 3 * D), lambda i: (0, 0, 0)),
       pl.BlockSpec((4, D), lambda i: (0, 0))],
    out_specs=pl.BlockSpec((R, D), lambda i: (i, 0)),
    out_shape=jax.ShapeDtypeStruct((N, D), jnp.float32))

_head = pl.pallas_call(
    _head_body,
    grid=(N // R,),
    in_specs=[pl.BlockSpec((R, D), lambda i: (i, 0)),
              pl.BlockSpec((D, 16), lambda i: (0, 0)),
              pl.BlockSpec((1, 16), lambda i: (0, 0)),
              pl.BlockSpec((16, 8), lambda i: (0, 0)),
              pl.BlockSpec((1, 8), lambda i: (0, 0)),
              pl.BlockSpec((8, 4), lambda i: (0, 0)),
              pl.BlockSpec((1, 4), lambda i: (0, 0)),
              pl.BlockSpec((4, 1), lambda i: (0, 0)),
              pl.BlockSpec((1, 1), lambda i: (0, 0))],
    out_specs=pl.BlockSpec((R, 1), lambda i: (i, 0)),
    out_shape=jax.ShapeDtypeStruct((N, 1), jnp.float32))


def kernel(x, edge_index, edge_weight, conv_W, conv_b, peep, gate_b,
           lw1, lb1, lw2, lb2, lw3, lb3, lw4, lb4):
    pad = EP - E
    srcp = jnp.concatenate(
        [edge_index[0], jnp.zeros((pad,), jnp.int32)]).reshape(NW, NCH, CH)
    dstp = jnp.concatenate(
        [edge_index[1], jnp.zeros((pad,), jnp.int32)]).reshape(NW, NCH, CH)
    wp = jnp.concatenate(
        [edge_weight, jnp.zeros((pad,), jnp.float32)]).reshape(NW, NCH, CH)

    degp = _deg_kernel(srcp, wp)
    dis = _dis_call(degp.reshape(NW, N)).reshape(N)
    nwp = _nw_kernel(dis, srcp, dstp, wp)

    def cell(X, l):
        o = 8 * l
        wcat = jnp.concatenate(
            [conv_W[o + 0], conv_W[o + 4], conv_W[o + 6]], axis=-1)
        bi = conv_b[o + 0] + conv_b[o + 1] + gate_b[l, 0, 0]
        bc = conv_b[o + 4] + conv_b[o + 5] + gate_b[l, 2, 0]
        bo = conv_b[o + 6] + conv_b[o + 7] + gate_b[l, 3, 0]
        bm = jnp.stack([bi, bc, bo, peep[l, 2, 0]])
        tx0 = X
        p = _prop_kernel(tx0, srcp, dstp, nwp)
        tx1 = _comb1(p)
        p = _prop_kernel(tx1, srcp, dstp, nwp)
        tx2 = _comb2(p, tx0)
        p = _prop_kernel(tx2, srcp, dstp, nwp)
        tx3 = _comb2(p, tx1)
        p = _prop_kernel(tx3, srcp, dstp, nwp)
        tx4 = _comb2(p, tx2)
        return _gates(tx0, tx1, tx2, tx3, tx4, wcat, bm)

    h = cell(x, 0)
    h = cell(h, 1)
    return _head(h, lw1, lb1.reshape(1, 16), lw2, lb2.reshape(1, 8),
                 lw3, lb3.reshape(1, 4), lw4, lb4.reshape(1, 1))
